# Initial kernel scaffold; baseline (speedup 1.0000x reference)
import jax
import jax.numpy as jnp
from jax.experimental import pallas as pl


def kernel(x, token_table, pos_table):
    raise NotImplementedError("write your pallas kernel here")


# speedup vs baseline: 2.1699x; 2.1699x over previous
import functools

import jax
import jax.numpy as jnp
from jax import lax
from jax.experimental import pallas as pl
from jax.experimental.pallas import tpu as pltpu
from jax.experimental.pallas import tpu_sc as plsc

_MAXLEN = 200
_D = 64
_B = 4096
_NC, _NS = 2, 16
_NW = _NC * _NS
_TOTAL = _B * _MAXLEN
_PER_W = _TOTAL // _NW
_G = 128
_NG = _PER_W // _G


def _body(tok_hbm, idx_hbm, pos_hbm, out_hbm, idx_v, pos_v, buf, sem):
    wid = lax.axis_index("s") * _NC + lax.axis_index("c")
    pltpu.sync_copy(idx_hbm.at[wid], idx_v)
    pltpu.sync_copy(pos_hbm, pos_v)

    def step(j, carry):
        pltpu.async_copy(tok_hbm.at[idx_v.at[j]], buf, sem).wait()
        off = lax.rem(j * _G, _MAXLEN)

        def add_row(i, c2):
            p = off + i
            for c in range(_D // 16):
                sl = pl.ds(c * 16, 16)
                buf[i, sl] = buf[i, sl] + pos_v[p, sl]
            return c2

        lax.fori_loop(0, _G, add_row, 0)
        pltpu.sync_copy(buf, out_hbm.at[wid, j])
        return carry

    lax.fori_loop(0, _NG, step, 0)


_emb = functools.partial(
    pl.kernel,
    out_type=jax.ShapeDtypeStruct((_NW, _NG, _G, _D), jnp.float32),
    mesh=plsc.VectorSubcoreMesh(
        core_axis_name="c", subcore_axis_name="s",
        num_cores=_NC, num_subcores=_NS),
    scratch_types=[
        pltpu.VMEM((_NG, _G), jnp.int32),
        pltpu.VMEM((2 * _MAXLEN, _D), jnp.float32),
        pltpu.VMEM((_G, _D), jnp.float32),
        pltpu.SemaphoreType.DMA,
    ],
    compiler_params=pltpu.CompilerParams(use_tc_tiling_on_sc=False),
)(_body)


def kernel(x, token_table, pos_table):
    idx = x.reshape(_NW, _NG, _G)
    pos2 = jnp.concatenate([pos_table, pos_table], axis=0)
    out = _emb(token_table, idx, pos2)
    return out.reshape(_B, _MAXLEN, _D)


# speedup vs baseline: 2.7632x; 1.2734x over previous
import functools

import jax
import jax.numpy as jnp
from jax import lax
from jax.experimental import pallas as pl
from jax.experimental.pallas import tpu as pltpu
from jax.experimental.pallas import tpu_sc as plsc

_MAXLEN = 200
_D = 64
_B = 4096
_NC, _NS = 2, 16
_NW = _NC * _NS
_TOTAL = _B * _MAXLEN
_PER_W = _TOTAL // _NW
_G = 128
_NG = _PER_W // _G
_NBUF = 4
_NROUND = _NG // _NBUF


def _body(tok_hbm, idx_hbm, pos_hbm, out_hbm, idx_v, pos_v,
          bi0, bi1, bi2, bi3, bo0, bo1, bo2, bo3,
          g0, g1, g2, g3, o0, o1, o2, o3):
    bins = [bi0, bi1, bi2, bi3]
    bouts = [bo0, bo1, bo2, bo3]
    gsems = [g0, g1, g2, g3]
    osems = [o0, o1, o2, o3]

    wid = lax.axis_index("s") * _NC + lax.axis_index("c")
    pltpu.sync_copy(idx_hbm.at[wid], idx_v)
    pltpu.sync_copy(pos_hbm, pos_v)

    def fire_gather(j, b):
        pltpu.async_copy(tok_hbm.at[idx_v.at[j]], bins[b], gsems[b])

    def wait_gather(j, b):
        pltpu.make_async_copy(tok_hbm.at[idx_v.at[j]], bins[b], gsems[b]).wait()

    def fire_write(j, b):
        pltpu.async_copy(bouts[b], out_hbm.at[wid, j], osems[b])

    def wait_write(j, b):
        pltpu.make_async_copy(bouts[b], out_hbm.at[wid, j], osems[b]).wait()

    def compute(j, b):
        off = lax.rem(j * _G, _MAXLEN)

        def add_row(i, c2):
            p = off + i
            for c in range(_D // 16):
                sl = pl.ds(c * 16, 16)
                bouts[b][i, sl] = bins[b][i, sl] + pos_v[p, sl]
            return c2

        lax.fori_loop(0, _G, add_row, 0)

    for b in range(_NBUF):
        fire_gather(b, b)

    for b in range(_NBUF):
        wait_gather(b, b)
        compute(b, b)
        fire_write(b, b)
        fire_gather(_NBUF + b, b)

    def round_body(r, carry):
        j0 = r * _NBUF
        for b in range(_NBUF):
            j = j0 + b
            wait_gather(j, b)
            wait_write(j - _NBUF, b)
            compute(j, b)
            fire_write(j, b)
            fire_gather(j + _NBUF, b)
        return carry

    lax.fori_loop(1, _NROUND - 1, round_body, 0)

    j0 = (_NROUND - 1) * _NBUF
    for b in range(_NBUF):
        j = j0 + b
        wait_gather(j, b)
        wait_write(j - _NBUF, b)
        compute(j, b)
        fire_write(j, b)

    for b in range(_NBUF):
        wait_write(j0 + b, b)


_emb = functools.partial(
    pl.kernel,
    out_type=jax.ShapeDtypeStruct((_NW, _NG, _G, _D), jnp.float32),
    mesh=plsc.VectorSubcoreMesh(
        core_axis_name="c", subcore_axis_name="s",
        num_cores=_NC, num_subcores=_NS),
    scratch_types=(
        [pltpu.VMEM((_NG, _G), jnp.int32),
         pltpu.VMEM((2 * _MAXLEN, _D), jnp.float32)]
        + [pltpu.VMEM((_G, _D), jnp.float32) for _ in range(2 * _NBUF)]
        + [pltpu.SemaphoreType.DMA for _ in range(2 * _NBUF)]
    ),
    compiler_params=pltpu.CompilerParams(use_tc_tiling_on_sc=False),
)(_body)


def kernel(x, token_table, pos_table):
    idx = x.reshape(_NW, _NG, _G)
    pos2 = jnp.concatenate([pos_table, pos_table], axis=0)
    out = _emb(token_table, idx, pos2)
    return out.reshape(_B, _MAXLEN, _D)


# speedup vs baseline: 4.2148x; 1.5253x over previous
import functools

import jax
import jax.numpy as jnp
from jax import lax
from jax.experimental import pallas as pl
from jax.experimental.pallas import tpu as pltpu
from jax.experimental.pallas import tpu_sc as plsc

_MAXLEN = 200
_D = 64
_B = 4096
_NC, _NS = 2, 16
_NW = _NC * _NS
_TOTAL = _B * _MAXLEN
_PER_W = _TOTAL // _NW
_G = 128
_NG = _PER_W // _G
_NBUF = 4
_NROUND = _NG // _NBUF


def _body(tok_hbm, idx_hbm, pos_hbm, out_hbm, idx_v, pos_v,
          bi0, bi1, bi2, bi3, bo0, bo1, bo2, bo3,
          g0, g1, g2, g3, o0, o1, o2, o3):
    bins = [bi0, bi1, bi2, bi3]
    bouts = [bo0, bo1, bo2, bo3]
    gsems = [g0, g1, g2, g3]
    osems = [o0, o1, o2, o3]

    wid = lax.axis_index("s") * _NC + lax.axis_index("c")
    pltpu.sync_copy(idx_hbm.at[wid], idx_v)
    pltpu.sync_copy(pos_hbm, pos_v)

    def fire_gather(j, b):
        pltpu.async_copy(tok_hbm.at[idx_v.at[j]], bins[b], gsems[b])

    def wait_gather(j, b):
        pltpu.make_async_copy(tok_hbm.at[idx_v.at[j]], bins[b], gsems[b]).wait()

    def fire_write(j, b):
        pltpu.async_copy(bouts[b], out_hbm.at[wid, j], osems[b])

    def wait_write(j, b):
        pltpu.make_async_copy(bouts[b], out_hbm.at[wid, j], osems[b]).wait()

    def compute(j, b):
        off = lax.rem(j * _G, _MAXLEN)

        @plsc.parallel_loop(0, _G, step=1, unroll=8)
        def add_row(i):
            p = off + i
            for c in range(_D // 16):
                sl = pl.ds(c * 16, 16)
                bouts[b][i, sl] = bins[b][i, sl] + pos_v[p, sl]

    for b in range(_NBUF):
        fire_gather(b, b)

    for b in range(_NBUF):
        wait_gather(b, b)
        compute(b, b)
        fire_write(b, b)
        fire_gather(_NBUF + b, b)

    def round_body(r, carry):
        j0 = r * _NBUF
        for b in range(_NBUF):
            j = j0 + b
            wait_gather(j, b)
            wait_write(j - _NBUF, b)
            compute(j, b)
            fire_write(j, b)
            fire_gather(j + _NBUF, b)
        return carry

    lax.fori_loop(1, _NROUND - 1, round_body, 0)

    j0 = (_NROUND - 1) * _NBUF
    for b in range(_NBUF):
        j = j0 + b
        wait_gather(j, b)
        wait_write(j - _NBUF, b)
        compute(j, b)
        fire_write(j, b)

    for b in range(_NBUF):
        wait_write(j0 + b, b)


_emb = functools.partial(
    pl.kernel,
    out_type=jax.ShapeDtypeStruct((_NW, _NG, _G, _D), jnp.float32),
    mesh=plsc.VectorSubcoreMesh(
        core_axis_name="c", subcore_axis_name="s",
        num_cores=_NC, num_subcores=_NS),
    scratch_types=(
        [pltpu.VMEM((_NG, _G), jnp.int32),
         pltpu.VMEM((2 * _MAXLEN, _D), jnp.float32)]
        + [pltpu.VMEM((_G, _D), jnp.float32) for _ in range(2 * _NBUF)]
        + [pltpu.SemaphoreType.DMA for _ in range(2 * _NBUF)]
    ),
    compiler_params=pltpu.CompilerParams(use_tc_tiling_on_sc=False),
)(_body)


def kernel(x, token_table, pos_table):
    idx = x.reshape(_NW, _NG, _G)
    pos2 = jnp.concatenate([pos_table, pos_table], axis=0)
    out = _emb(token_table, idx, pos2)
    return out.reshape(_B, _MAXLEN, _D)


# speedup vs baseline: 4.2412x; 1.0063x over previous
import functools

import jax
import jax.numpy as jnp
from jax import lax
from jax.experimental import pallas as pl
from jax.experimental.pallas import tpu as pltpu
from jax.experimental.pallas import tpu_sc as plsc

_MAXLEN = 200
_D = 64
_B = 4096
_NC, _NS = 2, 16
_NW = _NC * _NS
_G = _B // _NW
_NG = _MAXLEN
_NBUF = 4
_NROUND = _NG // _NBUF


def _body(tok_hbm, idx_hbm, pos_hbm, out_hbm, idx_v, pos_v,
          bi0, bi1, bi2, bi3, bo0, bo1, bo2, bo3,
          g0, g1, g2, g3, o0, o1, o2, o3):
    bins = [bi0, bi1, bi2, bi3]
    bouts = [bo0, bo1, bo2, bo3]
    gsems = [g0, g1, g2, g3]
    osems = [o0, o1, o2, o3]

    wid = lax.axis_index("s") * _NC + lax.axis_index("c")
    base = wid * _G
    pltpu.sync_copy(idx_hbm.at[wid], idx_v)
    pltpu.sync_copy(pos_hbm, pos_v)

    def fire_gather(j, b):
        pltpu.async_copy(tok_hbm.at[idx_v.at[j]], bins[b], gsems[b])

    def wait_gather(j, b):
        pltpu.make_async_copy(tok_hbm.at[idx_v.at[j]], bins[b], gsems[b]).wait()

    def fire_write(j, b):
        pltpu.async_copy(bouts[b], out_hbm.at[pl.ds(base, _G), j], osems[b])

    def wait_write(j, b):
        pltpu.make_async_copy(
            bouts[b], out_hbm.at[pl.ds(base, _G), j], osems[b]).wait()

    def compute(j, b):
        pvs = [pos_v[j, pl.ds(c * 16, 16)] for c in range(_D // 16)]

        @plsc.parallel_loop(0, _G, step=1, unroll=8)
        def add_row(i):
            for c in range(_D // 16):
                sl = pl.ds(c * 16, 16)
                bouts[b][i, sl] = bins[b][i, sl] + pvs[c]

    for b in range(_NBUF):
        fire_gather(b, b)

    for b in range(_NBUF):
        wait_gather(b, b)
        compute(b, b)
        fire_write(b, b)
        fire_gather(_NBUF + b, b)

    def round_body(r, carry):
        j0 = r * _NBUF
        for b in range(_NBUF):
            j = j0 + b
            wait_gather(j, b)
            wait_write(j - _NBUF, b)
            compute(j, b)
            fire_write(j, b)
            fire_gather(j + _NBUF, b)
        return carry

    lax.fori_loop(1, _NROUND - 1, round_body, 0)

    j0 = (_NROUND - 1) * _NBUF
    for b in range(_NBUF):
        j = j0 + b
        wait_gather(j, b)
        wait_write(j - _NBUF, b)
        compute(j, b)
        fire_write(j, b)

    for b in range(_NBUF):
        wait_write(j0 + b, b)


_emb = functools.partial(
    pl.kernel,
    out_type=jax.ShapeDtypeStruct((_B, _MAXLEN, _D), jnp.float32),
    mesh=plsc.VectorSubcoreMesh(
        core_axis_name="c", subcore_axis_name="s",
        num_cores=_NC, num_subcores=_NS),
    scratch_types=(
        [pltpu.VMEM((_NG, _G), jnp.int32),
         pltpu.VMEM((_MAXLEN, _D), jnp.float32)]
        + [pltpu.VMEM((_G, _D), jnp.float32) for _ in range(2 * _NBUF)]
        + [pltpu.SemaphoreType.DMA for _ in range(2 * _NBUF)]
    ),
    compiler_params=pltpu.CompilerParams(use_tc_tiling_on_sc=False),
)(_body)


def kernel(x, token_table, pos_table):
    idx = x.T.reshape(_NG, _NW, _G).swapaxes(0, 1)
    return _emb(token_table, idx, pos_table)
